# baseline (speedup 1.0000x reference)
import functools

import jax
import jax.numpy as jnp
from jax import lax
from jax.experimental import pallas as pl
from jax.experimental.pallas import tpu as pltpu
from jax.experimental.pallas import tpu_sc as plsc

EMBED_DIM = 64
LANES = 16
BT = 128
NBUF = 2


@functools.lru_cache(maxsize=None)
def _make_depad(vocab: int):
    info = plsc.get_sparse_core_info()
    num_workers = info.num_cores * info.num_subcores
    R = 256
    total = vocab // R
    tail = vocab - total * R
    chunks = -(-total // num_workers)
    assert R % 16 == 0 and tail in (0, 64) and chunks >= 6
    assert chunks % 2 == 1

    mesh = plsc.VectorSubcoreMesh(core_axis_name="c", subcore_axis_name="s")

    @functools.partial(
        pl.kernel,
        out_type=jax.ShapeDtypeStruct((vocab // 2, 128), jnp.float32),
        mesh=mesh,
        compiler_params=pltpu.CompilerParams(
            use_tc_tiling_on_sc=True, needs_layout_passes=False
        ),
        scratch_types=[
            [pltpu.VMEM((EMBED_DIM, R), jnp.float32) for _ in range(2)],
            [pltpu.VMEM((R // 2, 128), jnp.float32) for _ in range(2)],
            pltpu.VMEM((EMBED_DIM, 64), jnp.float32),
            pltpu.VMEM((32, 128), jnp.float32),
            [pltpu.SemaphoreType.DMA for _ in range(2)],
            [pltpu.SemaphoreType.DMA for _ in range(2)],
        ],
    )
    def depad_kernel(tableT_hbm, tailT_hbm, out_hbm, ibufs, obufs, tibuf,
                     tobuf, isems, osems):
        w = lax.axis_index("s") * info.num_cores + lax.axis_index("c")

        def cid(i):
            return jnp.minimum(i * num_workers + w, total - 1)

        def fire_in(i, b):
            off = pl.multiple_of(cid(i) * R, 128)
            for c in range(EMBED_DIM):
                pltpu.async_copy(
                    tableT_hbm.at[c, pl.ds(off, R)], ibufs[b].at[c], isems[b]
                )

        def wait_in(b):
            pltpu.make_async_copy(
                tableT_hbm.at[:, pl.ds(0, R)], ibufs[b], isems[b]
            ).wait()

        def fire_out(i, b):
            pltpu.async_copy(
                obufs[b],
                out_hbm.at[pl.ds(pl.multiple_of(cid(i) * (R // 2), 8), R // 2)],
                osems[b],
            )

        def wait_out(b):
            pltpu.make_async_copy(
                obufs[b], out_hbm.at[pl.ds(0, R // 2)], osems[b]
            ).wait()

        iota16 = jax.lax.iota(jnp.int32, LANES)
        par64 = (iota16 & 1) * 64

        def pack_strip(ibuf, obuf, width):
            rvecs = [(iota16 >> 1) + 8 * j for j in range(width // LANES)]

            @pl.loop(0, EMBED_DIM, unroll=4)
            def _comp(c):
                cvec = par64 + c
                vals = [
                    ibuf[c, pl.ds(LANES * j, LANES)]
                    for j in range(width // LANES)
                ]
                for j in range(width // LANES):
                    plsc.store_scatter(obuf, [rvecs[j], cvec], vals[j])

        def bridge(b):
            pack_strip(ibufs[b], obufs[b], R)

        def step(i, b, do_wait_out, do_fire_in):
            wait_in(b)
            bridge(b)
            if do_wait_out:
                wait_out(b)
            fire_out(i, b)
            if do_fire_in:
                fire_in(i + 2, b)

        fire_in(0, 0)
        fire_in(1, 1)
        step(0, 0, False, True)
        step(1, 1, False, True)

        @pl.loop(2, chunks - 3, step=2)
        def _main(i):
            for b in range(2):
                step(i + b, b, True, True)

        step(chunks - 3, 0, True, True)
        step(chunks - 2, 1, True, False)
        step(chunks - 1, 0, True, False)
        wait_out(1)
        wait_out(0)

        @pl.when(w == num_workers - 1)
        def _tail():
            pltpu.sync_copy(tailT_hbm, tibuf)
            pack_strip(tibuf, tobuf, 64)
            pltpu.sync_copy(tobuf, out_hbm.at[pl.ds(total * (R // 2), 32)])

    def run_depad(table):
        tT = table.T
        return depad_kernel(tT, tT[:, total * R:])

    return run_depad


@functools.lru_cache(maxsize=None)
def _make_gather(batch: int, seq: int):
    info = plsc.get_sparse_core_info()
    num_workers = info.num_cores * info.num_subcores
    assert batch // BT == num_workers
    n_bt = batch // BT
    n_ct = EMBED_DIM // 8

    mesh = plsc.VectorSubcoreMesh(core_axis_name="c", subcore_axis_name="s")

    @functools.partial(
        pl.kernel,
        out_type=jax.ShapeDtypeStruct((seq, n_ct, n_bt, 8, BT), jnp.float32),
        mesh=mesh,
        compiler_params=pltpu.CompilerParams(
            use_tc_tiling_on_sc=False, needs_layout_passes=False
        ),
        scratch_types=[
            pltpu.VMEM((seq, BT), jnp.int32),
            pltpu.VMEM((LANES, LANES), jnp.int32),
            [pltpu.VMEM((BT, EMBED_DIM), jnp.float32) for _ in range(NBUF)],
            [pltpu.VMEM((n_ct, 8, BT), jnp.float32) for _ in range(NBUF)],
            [pltpu.SemaphoreType.DMA for _ in range(NBUF)],
            [pltpu.SemaphoreType.DMA for _ in range(NBUF)],
        ],
    )
    def gather_kernel(idx_hbm, table_hbm, out_hbm, idx_v, pconst, rbufs,
                      tbufs, gsems, ssems):
        w = lax.axis_index("s") * info.num_cores + lax.axis_index("c")
        pltpu.sync_copy(idx_hbm.at[w], idx_v)

        def fire_gather(s, b):
            pltpu.async_copy(table_hbm.at[idx_v.at[s]], rbufs[b], gsems[b])

        def wait_gather(b):
            pltpu.make_async_copy(
                table_hbm.at[idx_v.at[0]], rbufs[b], gsems[b]
            ).wait()

        def fire_store(s, b):
            pltpu.async_copy(tbufs[b], out_hbm.at[s, :, w], ssems[b])

        def wait_store(b):
            pltpu.make_async_copy(
                tbufs[b], out_hbm.at[0, :, 0], ssems[b]
            ).wait()

        iota = jax.lax.iota(jnp.int32, LANES)
        rows = [iota + l * LANES for l in range(BT // LANES)]
        for d in range(LANES):
            pconst[d, :] = (iota + d) % LANES

        def transpose(b):
            rbuf, tbuf = rbufs[b], tbufs[b]

            @pl.loop(0, LANES)
            def _diag(d):
                base = pconst[d, :]
                for k in range(EMBED_DIM // LANES):
                    colv = base + (k * LANES)
                    d0 = colv >> 3
                    d1 = colv & 7
                    vals = [
                        plsc.load_gather(rbuf, [rows[l], colv])
                        for l in range(BT // LANES)
                    ]
                    for l in range(BT // LANES):
                        plsc.store_scatter(tbuf, [d0, d1, rows[l]], vals[l])

        for b in range(NBUF):
            fire_gather(b, b)
        for b in range(NBUF):
            wait_gather(b)
            transpose(b)
            fire_store(b, b)
            fire_gather(b + NBUF, b)

        @pl.loop(NBUF, seq - NBUF, step=NBUF)
        def _blk(s0):
            for b in range(NBUF):
                s = s0 + b
                wait_gather(b)
                wait_store(b)
                transpose(b)
                fire_store(s, b)
                fire_gather(s + NBUF, b)

        for b in range(NBUF):
            s = seq - NBUF + b
            wait_gather(b)
            wait_store(b)
            transpose(b)
            fire_store(s, b)
        for b in range(NBUF):
            wait_store(b)

    def run(x, table):
        vocab = table.shape[0]
        packed = _make_depad(vocab)(table)
        table_lin = packed.reshape(vocab, EMBED_DIM)
        xw = x.reshape(num_workers, BT, seq).transpose(0, 2, 1)
        out5 = gather_kernel(xw, table_lin)
        return out5.transpose(2, 4, 0, 1, 3).reshape(batch, seq, EMBED_DIM)

    return run


def kernel(x, embedding):
    b0, b1 = x.shape
    return _make_gather(b0, b1)(x.astype(jnp.int32), embedding)


# speedup vs baseline: 1.4938x; 1.4938x over previous
import functools

import jax
import jax.numpy as jnp
from jax import lax
from jax.experimental import pallas as pl
from jax.experimental.pallas import tpu as pltpu
from jax.experimental.pallas import tpu_sc as plsc

EMBED_DIM = 64
LANES = 16
BT = 128
NBUF = 2


@functools.lru_cache(maxsize=None)
def _make_depad(vocab: int):
    info = plsc.get_sparse_core_info()
    num_workers = info.num_cores * info.num_subcores
    M = 80
    total = vocab // M
    chunks = -(-total // num_workers)
    assert total * M == vocab and M % 16 == 0 and chunks >= 6
    assert chunks % 2 == 1

    mesh = plsc.VectorSubcoreMesh(core_axis_name="c", subcore_axis_name="s")

    @functools.partial(
        pl.kernel,
        out_type=jax.ShapeDtypeStruct((vocab // 2, 128), jnp.float32),
        mesh=mesh,
        compiler_params=pltpu.CompilerParams(use_tc_tiling_on_sc=True),
        scratch_types=[
            [pltpu.VMEM((M, 64), jnp.float32) for _ in range(2)],
            [pltpu.VMEM((M // 2, 128), jnp.float32) for _ in range(2)],
            [pltpu.SemaphoreType.DMA for _ in range(2)],
            [pltpu.SemaphoreType.DMA for _ in range(2)],
        ],
    )
    def depad_kernel(table_hbm, out_hbm, ibufs, obufs, isems, osems):
        w = lax.axis_index("s") * info.num_cores + lax.axis_index("c")

        def cid(i):
            return jnp.minimum(i * num_workers + w, total - 1)

        def fire_in(i, b):
            pltpu.async_copy(
                table_hbm.at[pl.ds(pl.multiple_of(cid(i) * M, 16), M)],
                ibufs[b], isems[b],
            )

        def wait_in(b):
            pltpu.make_async_copy(
                table_hbm.at[pl.ds(0, M)], ibufs[b], isems[b]
            ).wait()

        def fire_out(i, b):
            pltpu.async_copy(
                obufs[b],
                out_hbm.at[pl.ds(pl.multiple_of(cid(i) * (M // 2), 8), M // 2)],
                osems[b],
            )

        def wait_out(b):
            pltpu.make_async_copy(
                obufs[b], out_hbm.at[pl.ds(0, M // 2)], osems[b]
            ).wait()

        def bridge(b):
            ibuf, obuf = ibufs[b], obufs[b]

            @pl.loop(0, M // 2, unroll=4)
            def _row(p):
                for q in range(8):
                    obuf[p, pl.ds(q * 16, 16)] = ibuf[
                        2 * p + q // 4, pl.ds((q % 4) * 16, 16)
                    ]

        def step(i, b, do_wait_out, do_fire_in):
            wait_in(b)
            bridge(b)
            if do_wait_out:
                wait_out(b)
            fire_out(i, b)
            if do_fire_in:
                fire_in(i + 2, b)

        fire_in(0, 0)
        fire_in(1, 1)
        step(0, 0, False, True)
        step(1, 1, False, True)

        @pl.loop(2, chunks - 3, step=2)
        def _main(i):
            for b in range(2):
                step(i + b, b, True, True)

        step(chunks - 3, 0, True, True)
        step(chunks - 2, 1, True, False)
        step(chunks - 1, 0, True, False)
        wait_out(1)
        wait_out(0)

    return depad_kernel


@functools.lru_cache(maxsize=None)
def _make_gather(batch: int, seq: int):
    info = plsc.get_sparse_core_info()
    num_workers = info.num_cores * info.num_subcores
    assert batch // BT == num_workers
    n_bt = batch // BT
    n_ct = EMBED_DIM // 8

    mesh = plsc.VectorSubcoreMesh(core_axis_name="c", subcore_axis_name="s")

    @functools.partial(
        pl.kernel,
        out_type=jax.ShapeDtypeStruct((seq, n_ct, n_bt, 8, BT), jnp.float32),
        mesh=mesh,
        compiler_params=pltpu.CompilerParams(
            use_tc_tiling_on_sc=False, needs_layout_passes=False
        ),
        scratch_types=[
            pltpu.VMEM((seq, BT), jnp.int32),
            pltpu.VMEM((LANES, LANES), jnp.int32),
            [pltpu.VMEM((BT, EMBED_DIM), jnp.float32) for _ in range(NBUF)],
            [pltpu.VMEM((n_ct, 8, BT), jnp.float32) for _ in range(NBUF)],
            [pltpu.SemaphoreType.DMA for _ in range(NBUF)],
            [pltpu.SemaphoreType.DMA for _ in range(NBUF)],
        ],
    )
    def gather_kernel(idx_hbm, table_hbm, out_hbm, idx_v, pconst, rbufs,
                      tbufs, gsems, ssems):
        w = lax.axis_index("s") * info.num_cores + lax.axis_index("c")
        pltpu.sync_copy(idx_hbm.at[w], idx_v)

        def fire_gather(s, b):
            pltpu.async_copy(table_hbm.at[idx_v.at[s]], rbufs[b], gsems[b])

        def wait_gather(b):
            pltpu.make_async_copy(
                table_hbm.at[idx_v.at[0]], rbufs[b], gsems[b]
            ).wait()

        def fire_store(s, b):
            pltpu.async_copy(tbufs[b], out_hbm.at[s, :, w], ssems[b])

        def wait_store(b):
            pltpu.make_async_copy(
                tbufs[b], out_hbm.at[0, :, 0], ssems[b]
            ).wait()

        iota = jax.lax.iota(jnp.int32, LANES)
        rows = [iota + l * LANES for l in range(BT // LANES)]
        for d in range(LANES):
            pconst[d, :] = (iota + d) % LANES

        def transpose(b):
            rbuf, tbuf = rbufs[b], tbufs[b]

            @pl.loop(0, LANES)
            def _diag(d):
                base = pconst[d, :]
                for k in range(EMBED_DIM // LANES):
                    colv = base + (k * LANES)
                    d0 = colv >> 3
                    d1 = colv & 7
                    vals = [
                        plsc.load_gather(rbuf, [rows[l], colv])
                        for l in range(BT // LANES)
                    ]
                    for l in range(BT // LANES):
                        plsc.store_scatter(tbuf, [d0, d1, rows[l]], vals[l])

        for b in range(NBUF):
            fire_gather(b, b)
        for b in range(NBUF):
            wait_gather(b)
            transpose(b)
            fire_store(b, b)
            fire_gather(b + NBUF, b)

        @pl.loop(NBUF, seq - NBUF, step=NBUF)
        def _blk(s0):
            for b in range(NBUF):
                s = s0 + b
                wait_gather(b)
                wait_store(b)
                transpose(b)
                fire_store(s, b)
                fire_gather(s + NBUF, b)

        for b in range(NBUF):
            s = seq - NBUF + b
            wait_gather(b)
            wait_store(b)
            transpose(b)
            fire_store(s, b)
        for b in range(NBUF):
            wait_store(b)

    def run(x, table):
        vocab = table.shape[0]
        packed = _make_depad(vocab)(table)
        table_lin = packed.reshape(vocab, EMBED_DIM)
        xw = x.reshape(num_workers, BT, seq).transpose(0, 2, 1)
        out5 = gather_kernel(xw, table_lin)
        return out5.transpose(2, 4, 0, 1, 3).reshape(batch, seq, EMBED_DIM)

    return run


def kernel(x, embedding):
    b0, b1 = x.shape
    return _make_gather(b0, b1)(x.astype(jnp.int32), embedding)
